# baseline (speedup 1.0000x reference)
import functools

import jax
import jax.numpy as jnp
from jax import lax
from jax.experimental import pallas as pl
from jax.experimental.pallas import tpu as pltpu
from jax.experimental.pallas import tpu_sc as plsc

NC = 2
NS = 16
NW = NC * NS

CHUNK = 128
NBUF = 4


def _gather_kernel(n_chunks_per_worker, d):
    chunks_pw = n_chunks_per_worker
    assert chunks_pw % NBUF == 0
    rounds = chunks_pw // NBUF

    mesh = plsc.VectorSubcoreMesh(core_axis_name="c", subcore_axis_name="s")

    def body(idx_hbm, table_hbm, out_hbm, idx_v, rows_v, *sems):
        gsems = sems[:NBUF]
        wsems = sems[NBUF:]
        wid = lax.axis_index("s") * NC + lax.axis_index("c")
        cbase = wid * chunks_pw

        pltpu.sync_copy(idx_hbm.at[pl.ds(cbase, chunks_pw)], idx_v)

        def start_gather(g, b):
            pltpu.async_copy(table_hbm.at[idx_v.at[g]], rows_v.at[b], gsems[b])

        def wait_gather(g, b):
            pltpu.make_async_copy(
                table_hbm.at[idx_v.at[g]], rows_v.at[b], gsems[b]
            ).wait()

        def start_write(g, b):
            off = (cbase + g) * CHUNK
            pltpu.async_copy(rows_v.at[b], out_hbm.at[pl.ds(off, CHUNK)], wsems[b])

        def wait_write(g, b):
            off = (cbase + g) * CHUNK
            pltpu.make_async_copy(
                rows_v.at[b], out_hbm.at[pl.ds(off, CHUNK)], wsems[b]
            ).wait()

        for b in range(NBUF):
            start_gather(b, b)

        def round_body(r, carry):
            for b in range(NBUF):
                g = r * NBUF + b
                wait_gather(g, b)
                start_write(g, b)
            for b in range(NBUF):
                g = r * NBUF + b
                wait_write(g, b)
                start_gather(g + NBUF, b)
            return carry

        lax.fori_loop(0, rounds - 1, round_body, 0)

        r = rounds - 1
        for b in range(NBUF):
            g = r * NBUF + b
            wait_gather(g, b)
            start_write(g, b)
        for b in range(NBUF):
            g = r * NBUF + b
            wait_write(g, b)

    n_rows = chunks_pw * NW * CHUNK
    return pl.kernel(
        body,
        out_type=jax.ShapeDtypeStruct((n_rows, d), jnp.float32),
        mesh=mesh,
        scratch_types=[
            pltpu.VMEM((chunks_pw, CHUNK), jnp.int32),
            pltpu.VMEM((NBUF, CHUNK, d), jnp.float32),
        ] + [pltpu.SemaphoreType.DMA] * (2 * NBUF),
        compiler_params=pltpu.CompilerParams(use_tc_tiling_on_sc=False),
    )


def kernel(token_ids, weight):
    batch, seq = token_ids.shape
    n_rows = batch * seq
    d = weight.shape[1]
    assert n_rows % (NW * CHUNK * NBUF) == 0
    chunks_pw = n_rows // (NW * CHUNK)
    idx = token_ids.reshape(n_rows // CHUNK, CHUNK).astype(jnp.int32)
    out = _gather_kernel(chunks_pw, d)(idx, weight)
    return out.reshape(batch, seq, d)
